# baseline (speedup 1.0000x reference)
import functools

import jax
import jax.numpy as jnp
from jax import lax
from jax.experimental import pallas as pl
from jax.experimental.pallas import tpu as pltpu
from jax.experimental.pallas import tpu_sc as plsc

_TEMP_INV = 20.0
_B = 1024
_D = 64
_N = 100000
_BN = 2048
_NBLK = (_N + _BN - 1) // _BN

_NC = 2
_NS = 16
_NW = _NC * _NS
_BPW = _B // _NW
_L = 16


def _sc_pick_kernel(t_hbm, vt_hbm, out_hbm, tv, rows_v, pick_v, sem):
    w = lax.axis_index("s") * _NC + lax.axis_index("c")
    pltpu.sync_copy(t_hbm, tv)
    idx = tv.at[pl.ds(w * _BPW, _BPW)]
    pltpu.async_copy(vt_hbm.at[idx], rows_v, sem).wait()
    lane = lax.iota(jnp.int32, _L)
    for g in range(_BPW // _L):
        acc = jnp.zeros((_L,), jnp.float32)
        for i in range(_L):
            r = g * _L + i
            rv = rows_v[r, pl.ds(w * _BPW + g * _L, _L)]
            acc = jnp.where(lane == i, rv, acc)
        pick_v[pl.ds(g * _L, _L)] = acc
    pltpu.sync_copy(pick_v, out_hbm.at[pl.ds(w * _BPW, _BPW)])


def _sc_pick_call(targets_i32, out_t):
    mesh = plsc.VectorSubcoreMesh(core_axis_name="c", subcore_axis_name="s")
    run = functools.partial(
        pl.kernel,
        mesh=mesh,
        out_type=jax.ShapeDtypeStruct((_B,), jnp.float32),
        scratch_types=[
            pltpu.VMEM((_B,), jnp.int32),
            pltpu.VMEM((_BPW, _B), jnp.float32),
            pltpu.VMEM((_BPW,), jnp.float32),
            pltpu.SemaphoreType.DMA,
        ],
    )(_sc_pick_kernel)
    return run(targets_i32, out_t)


def _tc_kernel(x_ref, ft_ref, out_ref, se8_ref, xs_ref, acc_ref):
    j = pl.program_id(0)

    @pl.when(j == 0)
    def _init():
        x = x_ref[...]
        nrm = jnp.maximum(jnp.sqrt(jnp.sum(x * x, axis=1, keepdims=True)),
                          1e-12)
        xs_ref[...] = x * (_TEMP_INV / nrm)
        acc_ref[...] = jnp.zeros_like(acc_ref)

    vt = jax.lax.dot_general(
        ft_ref[...], xs_ref[...], (((0,), (1,)), ((), ())),
        preferred_element_type=jnp.float32)
    out_ref[...] = vt

    @pl.when(j < _NBLK - 1)
    def _accum():
        e = jnp.exp(vt)
        acc = acc_ref[...]
        for k in range(_BN // 8):
            acc = acc + e[k * 8:(k + 1) * 8, :]
        acc_ref[...] = acc

    @pl.when(j == _NBLK - 1)
    def _fin():
        row = (_NBLK - 1) * _BN + jax.lax.broadcasted_iota(
            jnp.int32, (_BN, _B), 0)
        e = jnp.where(row < _N, jnp.exp(vt), 0.0)
        acc = acc_ref[...]
        for k in range(_BN // 8):
            acc = acc + e[k * 8:(k + 1) * 8, :]
        se = jnp.sum(acc, axis=0, keepdims=True)
        se8_ref[...] = jnp.broadcast_to(se, (8, _B))


def _tc_call(inputs, features_t):
    return pl.pallas_call(
        _tc_kernel,
        grid=(_NBLK,),
        in_specs=[
            pl.BlockSpec((_B, _D), lambda j: (0, 0)),
            pl.BlockSpec((_D, _BN), lambda j: (0, j)),
        ],
        out_specs=[
            pl.BlockSpec((_BN, _B), lambda j: (j, 0)),
            pl.BlockSpec((8, _B), lambda j: (0, 0)),
        ],
        out_shape=[
            jax.ShapeDtypeStruct((_N, _B), jnp.float32),
            jax.ShapeDtypeStruct((8, _B), jnp.float32),
        ],
        scratch_shapes=[
            pltpu.VMEM((_B, _D), jnp.float32),
            pltpu.VMEM((8, _B), jnp.float32),
        ],
    )(inputs, features_t)


def _combine_kernel(pick_ref, se8_ref, loss_ref):
    lse = jnp.log(se8_ref[0:1, :])
    loss_ref[0, 0] = (jnp.sum(lse) - jnp.sum(pick_ref[...])) / _B


def _combine_call(picked8, se8):
    return pl.pallas_call(
        _combine_kernel,
        in_specs=[
            pl.BlockSpec((8, 128), lambda: (0, 0)),
            pl.BlockSpec((8, _B), lambda: (0, 0)),
        ],
        out_specs=pl.BlockSpec(memory_space=pltpu.SMEM),
        out_shape=jax.ShapeDtypeStruct((1, 1), jnp.float32),
    )(picked8, se8)


def kernel(inputs, targets, features):
    t32 = targets.astype(jnp.int32)
    out_t, se8 = _tc_call(inputs, features.T)
    picked = _sc_pick_call(t32, out_t)
    loss2d = _combine_call(picked.reshape(8, 128), se8)
    outputs = out_t.T
    loss = loss2d[0, 0]
    loss = jnp.where(jnp.isnan(loss), jnp.float32(0.0), loss)
    return (loss, outputs)


# speedup vs baseline: 1.0605x; 1.0605x over previous
import functools

import jax
import jax.numpy as jnp
from jax import lax
from jax.experimental import pallas as pl
from jax.experimental.pallas import tpu as pltpu
from jax.experimental.pallas import tpu_sc as plsc

_TEMP_INV = 20.0
_B = 1024
_D = 64
_N = 100000
_BN = 4096
_NBLK = (_N + _BN - 1) // _BN

_NC = 2
_NS = 16
_NW = _NC * _NS
_BPW = _B // _NW
_L = 16


def _sc_pick_kernel(t_hbm, vt_hbm, out_hbm, tv, rows_v, pick_v, sem):
    w = lax.axis_index("s") * _NC + lax.axis_index("c")
    pltpu.sync_copy(t_hbm, tv)
    idx = tv.at[pl.ds(w * _BPW, _BPW)]
    pltpu.async_copy(vt_hbm.at[idx], rows_v, sem).wait()
    lane = lax.iota(jnp.int32, _L)
    for g in range(_BPW // _L):
        acc = jnp.zeros((_L,), jnp.float32)
        for i in range(_L):
            r = g * _L + i
            rv = rows_v[r, pl.ds(w * _BPW + g * _L, _L)]
            acc = jnp.where(lane == i, rv, acc)
        pick_v[pl.ds(g * _L, _L)] = acc
    pltpu.sync_copy(pick_v, out_hbm.at[pl.ds(w * _BPW, _BPW)])


def _sc_pick_call(targets_i32, out_t):
    mesh = plsc.VectorSubcoreMesh(core_axis_name="c", subcore_axis_name="s")
    run = functools.partial(
        pl.kernel,
        mesh=mesh,
        out_type=jax.ShapeDtypeStruct((_B,), jnp.float32),
        scratch_types=[
            pltpu.VMEM((_B,), jnp.int32),
            pltpu.VMEM((_BPW, _B), jnp.float32),
            pltpu.VMEM((_BPW,), jnp.float32),
            pltpu.SemaphoreType.DMA,
        ],
    )(_sc_pick_kernel)
    return run(targets_i32, out_t)


def _tc_kernel(x_ref, ft_ref, out_ref, se8_ref, xs_ref, acc_ref):
    j = pl.program_id(0)

    @pl.when(j == 0)
    def _init():
        x = x_ref[...]
        nrm = jnp.maximum(jnp.sqrt(jnp.sum(x * x, axis=1, keepdims=True)),
                          1e-12)
        xs_ref[...] = x * (_TEMP_INV / nrm)
        acc_ref[...] = jnp.zeros_like(acc_ref)

    vt = jax.lax.dot_general(
        ft_ref[...], xs_ref[...], (((0,), (1,)), ((), ())),
        preferred_element_type=jnp.float32)
    out_ref[...] = vt

    @pl.when(j < _NBLK - 1)
    def _accum():
        acc = acc_ref[...]
        for k in range(_BN // 8):
            acc = acc + jnp.exp(out_ref[k * 8:(k + 1) * 8, :])
        acc_ref[...] = acc

    _LAST = _N - (_NBLK - 1) * _BN

    @pl.when(j == _NBLK - 1)
    def _fin():
        acc = acc_ref[...]
        for k in range(_LAST // 8):
            acc = acc + jnp.exp(out_ref[k * 8:(k + 1) * 8, :])
        se = jnp.sum(acc, axis=0, keepdims=True)
        se8_ref[...] = jnp.broadcast_to(se, (8, _B))


def _tc_call(inputs, features_t):
    return pl.pallas_call(
        _tc_kernel,
        grid=(_NBLK,),
        in_specs=[
            pl.BlockSpec((_B, _D), lambda j: (0, 0)),
            pl.BlockSpec((_D, _BN), lambda j: (0, j)),
        ],
        out_specs=[
            pl.BlockSpec((_BN, _B), lambda j: (j, 0)),
            pl.BlockSpec((8, _B), lambda j: (0, 0)),
        ],
        out_shape=[
            jax.ShapeDtypeStruct((_N, _B), jnp.float32),
            jax.ShapeDtypeStruct((8, _B), jnp.float32),
        ],
        scratch_shapes=[
            pltpu.VMEM((_B, _D), jnp.float32),
            pltpu.VMEM((8, _B), jnp.float32),
        ],
    )(inputs, features_t)


def _combine_kernel(pick_ref, se8_ref, loss_ref):
    lse = jnp.log(se8_ref[0:1, :])
    loss_ref[0, 0] = (jnp.sum(lse) - jnp.sum(pick_ref[...])) / _B


def _combine_call(picked8, se8):
    return pl.pallas_call(
        _combine_kernel,
        in_specs=[
            pl.BlockSpec((8, 128), lambda: (0, 0)),
            pl.BlockSpec((8, _B), lambda: (0, 0)),
        ],
        out_specs=pl.BlockSpec(memory_space=pltpu.SMEM),
        out_shape=jax.ShapeDtypeStruct((1, 1), jnp.float32),
    )(picked8, se8)


def kernel(inputs, targets, features):
    t32 = targets.astype(jnp.int32)
    out_t, se8 = _tc_call(inputs, features.T)
    picked = _sc_pick_call(t32, out_t)
    loss2d = _combine_call(picked.reshape(8, 128), se8)
    outputs = out_t.T
    loss = loss2d[0, 0]
    loss = jnp.where(jnp.isnan(loss), jnp.float32(0.0), loss)
    return (loss, outputs)


# speedup vs baseline: 1.0733x; 1.0121x over previous
import functools

import jax
import jax.numpy as jnp
from jax import lax
from jax.experimental import pallas as pl
from jax.experimental.pallas import tpu as pltpu
from jax.experimental.pallas import tpu_sc as plsc

_TEMP_INV = 20.0
_B = 1024
_D = 64
_N = 100000
_BN = 4096
_NBLK = (_N + _BN - 1) // _BN

_NC = 2
_NS = 16
_NW = _NC * _NS
_BPW = _B // _NW
_L = 16


def _sc_pick_kernel(t_hbm, vt_hbm, out_hbm, tv, rows_v, pick_v, sem):
    w = lax.axis_index("s") * _NC + lax.axis_index("c")
    pltpu.sync_copy(t_hbm, tv)
    idx = tv.at[pl.ds(w * _BPW, _BPW)]
    pltpu.async_copy(vt_hbm.at[idx], rows_v, sem).wait()
    lane = lax.iota(jnp.int32, _L)
    for g in range(_BPW // _L):
        acc = jnp.zeros((_L,), jnp.float32)
        for i in range(_L):
            r = g * _L + i
            rv = rows_v[r, pl.ds(w * _BPW + g * _L, _L)]
            acc = jnp.where(lane == i, rv, acc)
        pick_v[pl.ds(g * _L, _L)] = acc
    pltpu.sync_copy(pick_v, out_hbm.at[pl.ds(w * _BPW, _BPW)])


def _sc_pick_call(targets_i32, out_t):
    mesh = plsc.VectorSubcoreMesh(core_axis_name="c", subcore_axis_name="s")
    run = functools.partial(
        pl.kernel,
        mesh=mesh,
        out_type=jax.ShapeDtypeStruct((_B,), jnp.float32),
        scratch_types=[
            pltpu.VMEM((_B,), jnp.int32),
            pltpu.VMEM((_BPW, _B), jnp.float32),
            pltpu.VMEM((_BPW,), jnp.float32),
            pltpu.SemaphoreType.DMA,
        ],
    )(_sc_pick_kernel)
    return run(targets_i32, out_t)


def _tc_kernel(x_ref, ft_ref, out_ref, se8_ref, xs_ref, acc_ref):
    j = pl.program_id(0)

    @pl.when(j == 0)
    def _init():
        xt = x_ref[...]
        nrm = jnp.maximum(jnp.sqrt(jnp.sum(xt * xt, axis=0, keepdims=True)),
                          1e-12)
        xs_ref[...] = xt * (_TEMP_INV / nrm)
        acc_ref[...] = jnp.zeros_like(acc_ref)

    vt = jax.lax.dot_general(
        ft_ref[...], xs_ref[...], (((0,), (0,)), ((), ())),
        preferred_element_type=jnp.float32)
    out_ref[...] = vt

    @pl.when(j < _NBLK - 1)
    def _accum():
        acc = acc_ref[...]
        for k in range(_BN // 8):
            acc = acc + jnp.exp(out_ref[k * 8:(k + 1) * 8, :])
        acc_ref[...] = acc

    _LAST = _N - (_NBLK - 1) * _BN

    @pl.when(j == _NBLK - 1)
    def _fin():
        acc = acc_ref[...]
        for k in range(_LAST // 8):
            acc = acc + jnp.exp(out_ref[k * 8:(k + 1) * 8, :])
        se = jnp.sum(acc, axis=0, keepdims=True)
        se8_ref[...] = jnp.broadcast_to(se, (8, _B))


def _tc_call(inputs_t, features_t):
    return pl.pallas_call(
        _tc_kernel,
        grid=(_NBLK,),
        in_specs=[
            pl.BlockSpec((_D, _B), lambda j: (0, 0)),
            pl.BlockSpec((_D, _BN), lambda j: (0, j)),
        ],
        out_specs=[
            pl.BlockSpec((_BN, _B), lambda j: (j, 0)),
            pl.BlockSpec((8, _B), lambda j: (0, 0)),
        ],
        out_shape=[
            jax.ShapeDtypeStruct((_N, _B), jnp.float32),
            jax.ShapeDtypeStruct((8, _B), jnp.float32),
        ],
        scratch_shapes=[
            pltpu.VMEM((_D, _B), jnp.float32),
            pltpu.VMEM((8, _B), jnp.float32),
        ],
    )(inputs_t, features_t)


def _combine_kernel(pick_ref, se8_ref, loss_ref):
    lse = jnp.log(se8_ref[0:1, :])
    loss_ref[0, 0] = (jnp.sum(lse) - jnp.sum(pick_ref[...])) / _B


def _combine_call(picked8, se8):
    return pl.pallas_call(
        _combine_kernel,
        in_specs=[
            pl.BlockSpec((8, 128), lambda: (0, 0)),
            pl.BlockSpec((8, _B), lambda: (0, 0)),
        ],
        out_specs=pl.BlockSpec(memory_space=pltpu.SMEM),
        out_shape=jax.ShapeDtypeStruct((1, 1), jnp.float32),
    )(picked8, se8)


def kernel(inputs, targets, features):
    t32 = targets.astype(jnp.int32)
    out_t, se8 = _tc_call(inputs.T, features.T)
    picked = _sc_pick_call(t32, out_t)
    loss2d = _combine_call(picked.reshape(8, 128), se8)
    outputs = out_t.T
    loss = loss2d[0, 0]
    loss = jnp.where(jnp.isnan(loss), jnp.float32(0.0), loss)
    return (loss, outputs)


# speedup vs baseline: 1.0984x; 1.0234x over previous
import functools

import jax
import jax.numpy as jnp
from jax import lax
from jax.experimental import pallas as pl
from jax.experimental.pallas import tpu as pltpu
from jax.experimental.pallas import tpu_sc as plsc

_TEMP_INV = 20.0
_B = 1024
_D = 64
_N = 100000
_BN = 4096
_NBLK = (_N + _BN - 1) // _BN

_NC = 2
_NS = 16
_NW = _NC * _NS
_BPW = _B // _NW
_L = 16


def _sc_pick_kernel(t_hbm, vt_hbm, out_hbm, tv, rows_v, pick_v, sem):
    w = lax.axis_index("s") * _NC + lax.axis_index("c")
    pltpu.sync_copy(t_hbm, tv)
    idx = tv.at[pl.ds(w * _BPW, _BPW)]
    cbase = (w // 4) * 128
    coff = (w % 4) * _BPW
    pltpu.async_copy(vt_hbm.at[idx, pl.ds(cbase, 128)], rows_v, sem).wait()
    lane = lax.iota(jnp.int32, _L)
    for g in range(_BPW // _L):
        acc = jnp.zeros((_L,), jnp.float32)
        for i in range(_L):
            r = g * _L + i
            rv = rows_v[r, pl.ds(coff + g * _L, _L)]
            acc = jnp.where(lane == i, rv, acc)
        pick_v[pl.ds(g * _L, _L)] = acc
    pltpu.sync_copy(pick_v, out_hbm.at[pl.ds(w * _BPW, _BPW)])


def _sc_pick_call(targets_i32, out_t):
    mesh = plsc.VectorSubcoreMesh(core_axis_name="c", subcore_axis_name="s")
    run = functools.partial(
        pl.kernel,
        mesh=mesh,
        out_type=jax.ShapeDtypeStruct((_B,), jnp.float32),
        scratch_types=[
            pltpu.VMEM((_B,), jnp.int32),
            pltpu.VMEM((_BPW, 128), jnp.float32),
            pltpu.VMEM((_BPW,), jnp.float32),
            pltpu.SemaphoreType.DMA,
        ],
    )(_sc_pick_kernel)
    return run(targets_i32, out_t)


def _tc_kernel(x_ref, ft_ref, out_ref, se8_ref, xs_ref, acc_ref):
    j = pl.program_id(0)

    @pl.when(j == 0)
    def _init():
        xt = x_ref[...]
        nrm = jnp.maximum(jnp.sqrt(jnp.sum(xt * xt, axis=0, keepdims=True)),
                          1e-12)
        xs_ref[...] = xt * (_TEMP_INV / nrm)
        acc_ref[...] = jnp.zeros_like(acc_ref)

    vt = jax.lax.dot_general(
        ft_ref[...], xs_ref[...], (((0,), (0,)), ((), ())),
        preferred_element_type=jnp.float32)
    out_ref[...] = vt

    @pl.when(j < _NBLK - 1)
    def _accum():
        acc = acc_ref[...]
        for k in range(_BN // 8):
            acc = acc + jnp.exp(out_ref[k * 8:(k + 1) * 8, :])
        acc_ref[...] = acc

    _LAST = _N - (_NBLK - 1) * _BN

    @pl.when(j == _NBLK - 1)
    def _fin():
        acc = acc_ref[...]
        for k in range(_LAST // 8):
            acc = acc + jnp.exp(out_ref[k * 8:(k + 1) * 8, :])
        se = jnp.sum(acc, axis=0, keepdims=True)
        se8_ref[...] = jnp.broadcast_to(se, (8, _B))


def _tc_call(inputs_t, features_t):
    return pl.pallas_call(
        _tc_kernel,
        grid=(_NBLK,),
        in_specs=[
            pl.BlockSpec((_D, _B), lambda j: (0, 0)),
            pl.BlockSpec((_D, _BN), lambda j: (0, j)),
        ],
        out_specs=[
            pl.BlockSpec((_BN, _B), lambda j: (j, 0)),
            pl.BlockSpec((8, _B), lambda j: (0, 0)),
        ],
        out_shape=[
            jax.ShapeDtypeStruct((_N, _B), jnp.float32),
            jax.ShapeDtypeStruct((8, _B), jnp.float32),
        ],
        scratch_shapes=[
            pltpu.VMEM((_D, _B), jnp.float32),
            pltpu.VMEM((8, _B), jnp.float32),
        ],
    )(inputs_t, features_t)


def _combine_kernel(pick_ref, se8_ref, loss_ref):
    lse = jnp.log(se8_ref[0:1, :])
    loss_ref[0, 0] = (jnp.sum(lse) - jnp.sum(pick_ref[...])) / _B


def _combine_call(picked8, se8):
    return pl.pallas_call(
        _combine_kernel,
        in_specs=[
            pl.BlockSpec((8, 128), lambda: (0, 0)),
            pl.BlockSpec((8, _B), lambda: (0, 0)),
        ],
        out_specs=pl.BlockSpec(memory_space=pltpu.SMEM),
        out_shape=jax.ShapeDtypeStruct((1, 1), jnp.float32),
    )(picked8, se8)


def kernel(inputs, targets, features):
    t32 = targets.astype(jnp.int32)
    out_t, se8 = _tc_call(inputs.T, features.T)
    picked = _sc_pick_call(t32, out_t)
    loss2d = _combine_call(picked.reshape(8, 128), se8)
    outputs = out_t.T
    loss = loss2d[0, 0]
    loss = jnp.where(jnp.isnan(loss), jnp.float32(0.0), loss)
    return (loss, outputs)
